# baseline (speedup 1.0000x reference)
import functools

import jax
import jax.numpy as jnp
from jax import lax
from jax.experimental import pallas as pl
from jax.experimental.pallas import tpu as pltpu
from jax.experimental.pallas import tpu_sc as plsc

B, V, C = 1024, 20, 20
D = 64
DPAD = 128
RPB = V * C
NCHUNK = 5
CHUNK = RPB // NCHUNK
L = 16
NCOL = D // L


def _rsqrt16(x):
  i = lax.bitcast_convert_type(x, jnp.int32)
  y = lax.bitcast_convert_type(jnp.int32(0x5F3759DF) - (i >> 1), jnp.float32)
  y = y * (1.5 - 0.5 * x * y * y)
  y = y * (1.5 - 0.5 * x * y * y)
  return y


def _hsum16(v):
  dnums = lax.GatherDimensionNumbers(
      offset_dims=(), collapsed_slice_dims=(0,), start_index_map=(0,))
  for d in (8, 4, 2, 1):
    idx = lax.iota(jnp.int32, L) ^ d
    v = v + lax.gather(v, idx[:, None], dnums, slice_sizes=(1,),
                       mode=lax.GatherScatterMode.PROMISE_IN_BOUNDS)
  return v


def _scaled_row(rows_ref, r):
  vs = [rows_ref[r, pl.ds(L * c, L)] for c in range(NCOL)]
  ssv = vs[0] * vs[0] + vs[1] * vs[1] + vs[2] * vs[2] + vs[3] * vs[3]
  ssb = _hsum16(ssv)
  norm = ssb * _rsqrt16(ssb)
  scale = jnp.where(ssb > 1.0, 1.0 / (norm + 1e-7), 1.0)
  return [v * scale for v in vs]


def _make_kernel():
  info = plsc.get_sparse_core_info()
  nc, ns = info.num_cores, info.num_subcores
  nw = nc * ns
  bpw = B // nw
  ipw = bpw * RPB

  mesh = plsc.VectorSubcoreMesh(core_axis_name="c", subcore_axis_name="s")

  @functools.partial(
      pl.kernel,
      mesh=mesh,
      out_type=jax.ShapeDtypeStruct((B * V * D,), jnp.float32),
      compiler_params=pltpu.CompilerParams(use_tc_tiling_on_sc=False),
      scratch_types=[
          pltpu.VMEM((ipw,), jnp.int32),
          pltpu.VMEM((2, RPB, DPAD), jnp.float32),
          pltpu.VMEM((V * D,), jnp.float32),
          pltpu.SemaphoreType.DMA,
          pltpu.SemaphoreType.DMA,
      ],
  )
  def k(x_hbm, w_hbm, out_hbm, idx_v, rows_v, out_v, sem0, sem1):
    sems = (sem0, sem1)
    wid = lax.axis_index("s") * nc + lax.axis_index("c")
    base = wid * bpw

    pltpu.sync_copy(x_hbm.at[pl.ds(wid * ipw, ipw)], idx_v)

    def fire(b, slot):
      for j in range(NCHUNK):
        pltpu.async_copy(
            w_hbm.at[idx_v.at[pl.ds(b * RPB + j * CHUNK, CHUNK)]],
            rows_v.at[slot, pl.ds(j * CHUNK, CHUNK)],
            sems[slot],
        )

    def drain(slot):
      pltpu.make_async_copy(
          w_hbm.at[pl.ds(0, RPB)], rows_v.at[slot], sems[slot]
      ).wait()

    def compute(slot, b):
      rows = rows_v.at[slot]

      def head(r, carry):
        vs = _scaled_row(rows, r)
        for c in range(NCOL):
          out_v[pl.ds(r * D + L * c, L)] = vs[c]
        return carry

      lax.fori_loop(0, V - 1, head, 0)

      UNROLL = 4
      ngroups = (RPB - V) // UNROLL

      def tail4(g, acc):
        base_r = (V - 1) + UNROLL * g
        rvs = [_scaled_row(rows, base_r + u) for u in range(UNROLL)]
        return tuple(
            a + ((rvs[0][c] + rvs[1][c]) + (rvs[2][c] + rvs[3][c]))
            for c, a in enumerate(acc)
        )

      zero = jnp.zeros((L,), jnp.float32)
      acc = lax.fori_loop(0, ngroups, tail4, (zero,) * NCOL)
      vs_last = _scaled_row(rows, RPB - 1)
      acc = tuple(a + v for a, v in zip(acc, vs_last))
      for c in range(NCOL):
        out_v[pl.ds((V - 1) * D + L * c, L)] = acc[c]

      pltpu.sync_copy(out_v, out_hbm.at[pl.ds((base + b) * V * D, V * D)])

    fire(0, 0)
    fire(1, 1)

    def pair(g, carry):
      for s in range(2):
        b = 2 * g + s
        drain(s)
        compute(s, b)

        @pl.when(b + 2 < bpw)
        def _():
          fire(b + 2, s)

      return carry

    lax.fori_loop(0, bpw // 2, pair, 0)

  return k


_kernel = _make_kernel()


def kernel(x, weight):
  xf = x.astype(jnp.int32).reshape(B * V * C)
  wp = jnp.pad(weight, ((0, 0), (0, DPAD - D)))
  return _kernel(xf, wp).reshape(B, V, D)


# speedup vs baseline: 1.1478x; 1.1478x over previous
import functools

import jax
import jax.numpy as jnp
from jax import lax
from jax.experimental import pallas as pl
from jax.experimental.pallas import tpu as pltpu
from jax.experimental.pallas import tpu_sc as plsc

B, V, C = 1024, 20, 20
D = 64
RPB = V * C
NCHUNK = 5
CHUNK = RPB // NCHUNK
L = 16
NCOL = D // L

_DNUMS = lax.GatherDimensionNumbers(
    offset_dims=(), collapsed_slice_dims=(0,), start_index_map=(0,))


def _shuffle(v, idx):
  return lax.gather(v, idx[:, None], _DNUMS, slice_sizes=(1,),
                    mode=lax.GatherScatterMode.PROMISE_IN_BOUNDS)


def _scaled_rows4(rows_ref, r0):
  lanes = lax.iota(jnp.int32, L)
  data = [[rows_ref[r0 + u, pl.ds(L * c, L)] for c in range(NCOL)]
          for u in range(4)]
  ssv = [d[0] * d[0] + d[1] * d[1] + d[2] * d[2] + d[3] * d[3] for d in data]
  red = []
  for s in ssv:
    s = s + _shuffle(s, lanes ^ 8)
    s = s + _shuffle(s, lanes ^ 4)
    red.append(s)
  gid = lanes >> 2
  m = red[0]
  for k in range(1, 4):
    m = jnp.where(gid == k, red[k], m)
  m = m + _shuffle(m, lanes ^ 2)
  m = m + _shuffle(m, lanes ^ 1)
  i = lax.bitcast_convert_type(m, jnp.int32)
  y = lax.bitcast_convert_type(jnp.int32(0x5F3759DF) - (i >> 1), jnp.float32)
  xh = 0.5 * m
  y = y * (1.5 - xh * y * y)
  y = y * (1.5 - xh * y * y)
  scale = jnp.where(m > 1.0, y - 1e-7 * (y * y), 1.0)
  out = []
  for u in range(4):
    su = _shuffle(scale, jnp.full((L,), 4 * u, jnp.int32))
    out.append([d * su for d in data[u]])
  return out


def _make_kernel():
  info = plsc.get_sparse_core_info()
  nc, ns = info.num_cores, info.num_subcores
  nw = nc * ns
  bpw = B // nw
  ipw = bpw * RPB

  mesh = plsc.VectorSubcoreMesh(core_axis_name="c", subcore_axis_name="s")

  @functools.partial(
      pl.kernel,
      mesh=mesh,
      out_type=jax.ShapeDtypeStruct((B * V * D,), jnp.float32),
      compiler_params=pltpu.CompilerParams(use_tc_tiling_on_sc=False),
      scratch_types=[
          pltpu.VMEM((ipw,), jnp.int32),
          pltpu.VMEM((2, RPB, D), jnp.float32),
          pltpu.VMEM((V * D,), jnp.float32),
          pltpu.SemaphoreType.DMA,
          pltpu.SemaphoreType.DMA,
      ],
  )
  def k(x_hbm, w_hbm, out_hbm, idx_v, rows_v, out_v, sem0, sem1):
    sems = (sem0, sem1)
    wid = lax.axis_index("s") * nc + lax.axis_index("c")
    base = wid * bpw

    pltpu.sync_copy(x_hbm.at[pl.ds(wid * ipw, ipw)], idx_v)

    def fire(b, slot):
      for j in range(NCHUNK):
        pltpu.async_copy(
            w_hbm.at[idx_v.at[pl.ds(b * RPB + j * CHUNK, CHUNK)]],
            rows_v.at[slot, pl.ds(j * CHUNK, CHUNK)],
            sems[slot],
        )

    def drain(slot):
      pltpu.make_async_copy(
          w_hbm.at[pl.ds(0, RPB)], rows_v.at[slot], sems[slot]
      ).wait()

    def compute(slot, b):
      rows = rows_v.at[slot]

      def store_row(r, vs):
        for c in range(NCOL):
          out_v[pl.ds(r * D + L * c, L)] = vs[c]

      def head(g, carry):
        r0 = 4 * g
        quad = _scaled_rows4(rows, r0)
        for u in range(4):
          store_row(r0 + u, quad[u])
        return carry

      lax.fori_loop(0, 4, head, 0)

      quad = _scaled_rows4(rows, 16)
      for u in range(3):
        store_row(16 + u, quad[u])
      acc = quad[3]

      def tail(g, acc):
        quad = _scaled_rows4(rows, 20 + 4 * g)
        return tuple(
            a + ((quad[0][c] + quad[1][c]) + (quad[2][c] + quad[3][c]))
            for c, a in enumerate(acc)
        )

      acc = lax.fori_loop(0, (RPB - 20) // 4, tail, tuple(acc))
      for c in range(NCOL):
        out_v[pl.ds((V - 1) * D + L * c, L)] = acc[c]

      pltpu.sync_copy(out_v, out_hbm.at[pl.ds((base + b) * V * D, V * D)])

    fire(0, 0)
    fire(1, 1)

    def pair(g, carry):
      for s in range(2):
        b = 2 * g + s
        drain(s)
        compute(s, b)

        @pl.when(b + 2 < bpw)
        def _():
          fire(b + 2, s)

      return carry

    lax.fori_loop(0, bpw // 2, pair, 0)

  return k


_kernel = _make_kernel()


def kernel(x, weight):
  xf = x.astype(jnp.int32).reshape(B * V * C)
  return _kernel(xf, weight).reshape(B, V, D)
